# baseline (speedup 1.0000x reference)
import functools

import jax
import jax.numpy as jnp
from jax import lax
from jax.experimental import pallas as pl
from jax.experimental.pallas import tpu as pltpu
import jax.experimental.pallas.tpu_sc as plsc

H = W = 512
NPIX = H * W
NC, NS = 2, 16
NW = NC * NS
CHUNK = NPIX // NW
F32 = jnp.float32
I32 = jnp.int32


def _shz(x, s, axis):
    n = x.shape[axis]
    if s == 0:
        return x
    if axis == 0:
        if s > 0:
            return jnp.concatenate([x[s:, :], jnp.zeros((s, x.shape[1]), x.dtype)], 0)
        return jnp.concatenate([jnp.zeros((-s, x.shape[1]), x.dtype), x[: n + s, :]], 0)
    if s > 0:
        return jnp.concatenate([x[:, s:], jnp.zeros((x.shape[0], s), x.dtype)], 1)
    return jnp.concatenate([jnp.zeros((x.shape[0], -s), x.dtype), x[:, : n + s]], 1)


def _maxpool25_axis(x, axis):
    f2 = jnp.maximum(x, _shz(x, 1, axis))
    f4 = jnp.maximum(f2, _shz(f2, 2, axis))
    f8 = jnp.maximum(f4, _shz(f4, 4, axis))
    r13 = jnp.maximum(f8, _shz(f8, 5, axis))
    b2 = jnp.maximum(x, _shz(x, -1, axis))
    b4 = jnp.maximum(b2, _shz(b2, -2, axis))
    b8 = jnp.maximum(b4, _shz(b4, -4, axis))
    l13 = jnp.maximum(b8, _shz(b8, -5, axis))
    return jnp.maximum(r13, l13)


def _expand25(m):
    return _maxpool25_axis(_maxpool25_axis(m, 0), 1)


def _blur5_valid(m):
    r = m
    r = _shz(m, -2, 0) + _shz(m, -1, 0) + m + _shz(m, 1, 0) + _shz(m, 2, 0)
    r = _shz(r, -2, 1) + _shz(r, -1, 1) + r + _shz(r, 1, 1) + _shz(r, 2, 1)
    return (r / 25.0) * m


def _prep_body(src_ref, tgt_ref, ms_ref, mr_ref, hk_ref, p1_ref, p2_ref, be_ref):
    i2 = pl.program_id(0) * 2
    for side in range(2):
        img_ref = src_ref if side == 0 else tgt_ref
        msk_ref = ms_ref if side == 0 else mr_ref
        img = jnp.clip((img_ref[0] + 1.0) / 2.0, 0.0, 1.0) * 255.0
        m0 = msk_ref[0, 0]
        m1 = msk_ref[0, 1]
        m4 = msk_ref[0, 4]
        me = _expand25(msk_ref[0, 2] + msk_ref[0, 3]) * m1
        v = [jnp.clip(img[c], 0.0, 255.0).astype(I32) for c in range(3)]
        ve = [jnp.clip(img[c] * me, 0.0, 255.0).astype(I32) for c in range(3)]
        bits3 = ((m1 > 0).astype(I32) | ((m4 > 0).astype(I32) << 1)
                 | ((m0 > 0).astype(I32) << 2)) << 8
        beye = (me > 0).astype(I32) << 8
        isb = (i2 + side) * 3
        for ch in range(3):
            kc = (bits3 | v[ch]) + (isb + ch) * 2048
            ke = (beye | ve[ch]) + (NCOMB + (isb + ch) * 512)
            hk_ref[0, side, ch] = kc[:256, :] | (kc[256:, :] << 16)
            hk_ref[0, side, 3 + ch] = ke[:256, :] | (ke[256:, :] << 16)
        if side == 0:
            p1_ref[0] = v[0] | (v[1] << 8) | (v[2] << 16)
            p2_ref[0] = ve[0] | (ve[1] << 8) | (ve[2] << 16)
            be_ref[0] = _blur5_valid(me)


def _prep(sources, targets, mask_srcs, mask_tars):
    return pl.pallas_call(
        _prep_body,
        grid=(4,),
        in_specs=[
            pl.BlockSpec((1, 3, H, W), lambda i: (i, 0, 0, 0)),
            pl.BlockSpec((1, 3, H, W), lambda i: (i, 0, 0, 0)),
            pl.BlockSpec((1, 5, H, W), lambda i: (i, 0, 0, 0)),
            pl.BlockSpec((1, 5, H, W), lambda i: (i, 0, 0, 0)),
        ],
        out_specs=[
            pl.BlockSpec((1, 2, 6, H // 2, W), lambda i: (i, 0, 0, 0, 0)),
            pl.BlockSpec((1, H, W), lambda i: (i, 0, 0)),
            pl.BlockSpec((1, H, W), lambda i: (i, 0, 0)),
            pl.BlockSpec((1, H, W), lambda i: (i, 0, 0)),
        ],
        out_shape=[
            jax.ShapeDtypeStruct((4, 2, 6, H // 2, W), I32),
            jax.ShapeDtypeStruct((4, H, W), I32),
            jax.ShapeDtypeStruct((4, H, W), I32),
            jax.ShapeDtypeStruct((4, H, W), F32),
        ],
    )(sources, targets, mask_srcs, mask_tars)


NCOMB = 4 * 2 * 3 * 8 * 256
NEYE = 4 * 2 * 3 * 2 * 256
HIST_W = NCOMB + NEYE


NKW = NPIX // 2
KCH = NKW // NW
SIDE_W = 6 * KCH


@functools.cache
def _hist_sc_kernel():
    mesh = plsc.VectorSubcoreMesh(core_axis_name="c", subcore_axis_name="s")
    return pl.kernel(
        _hist_sc_body,
        out_type=jax.ShapeDtypeStruct((NW * HIST_W,), F32),
        mesh=mesh,
        scratch_types=[
            pltpu.VMEM((SIDE_W,), I32),
            pltpu.VMEM((SIDE_W,), I32),
            pltpu.VMEM((HIST_W,), F32),
            pltpu.SemaphoreType.DMA,
        ],
        compiler_params=pltpu.CompilerParams(needs_layout_passes=False),
    )


def _hist_sc_body(hk_hbm, out_hbm, bufa, bufb, hist, sem):
    wid = lax.axis_index("s") * NC + lax.axis_index("c")
    zeros = jnp.zeros((16,), F32)
    ones = jnp.full((16,), 1.0, F32)
    bufs = [bufa, bufb]

    def issue(k, buf):
        descs = []
        for kt in range(6):
            off = (k * 6 + kt) * NKW + wid * KCH
            descs.append(pltpu.async_copy(
                hk_hbm.at[pl.ds(off, KCH)], buf.at[pl.ds(kt * KCH, KCH)], sem))
        return descs

    descs = issue(0, bufs[0])

    @pl.loop(0, HIST_W // 16, unroll=8)
    def _(i):
        hist[pl.ds(i * 16, 16)] = zeros

    for k in range(8):
        buf = bufs[k % 2]
        for d in descs:
            d.wait()
        if k < 7:
            descs = issue(k + 1, bufs[(k + 1) % 2])

        @pl.loop(0, SIDE_W // 16, unroll=8)
        def _(i):
            w = buf[pl.ds(i * 16, 16)]
            plsc.addupdate_scatter(hist, [w & 0xFFFF], ones)
            plsc.addupdate_scatter(hist, [lax.shift_right_logical(w, 16)], ones)

    pltpu.sync_copy(hist, out_hbm.at[pl.ds(wid * HIST_W, HIST_W)])


def _tables_body(comb_ref, eye_ref, tb_ref, te_ref):
    hc = jnp.sum(comb_ref[...], axis=0)
    he = jnp.sum(eye_ref[...], axis=0)[:, 256:512]
    combos = [hc[:, b * 256:(b + 1) * 256] for b in range(8)]
    hists = [
        combos[1] + combos[3] + combos[5] + combos[7],
        combos[2] + combos[3] + combos[6] + combos[7],
        combos[4] + combos[5] + combos[6] + combos[7],
        he,
    ]
    jj = lax.broadcasted_iota(I32, (256, 256), 0)
    ii = lax.broadcasted_iota(I32, (256, 256), 1)
    tri = (jj <= ii).astype(F32)
    tabs = []
    for h in hists:
        cs = lax.dot_general(h, tri, (((1,), (0,)), ((), ())),
                             precision=lax.Precision.HIGHEST,
                             preferred_element_type=F32)
        csn = cs / jnp.maximum(cs[:, 255:256], 1.0)
        tm = []
        for img in range(4):
            a = csn[img * 6: img * 6 + 3]
            b = csn[img * 6 + 3: img * 6 + 6]
            aa = jnp.broadcast_to(a[:, :, None], (3, 256, 256))
            bb = jnp.broadcast_to(b[:, None, :], (3, 256, 256))
            cnt = jnp.sum((bb < aa).astype(F32), axis=2)
            tm.append(jnp.clip(cnt, 0.0, 255.0).astype(I32))
        tabs.append(tm)
    for img in range(4):
        tb_ref[img] = tabs[0][img] | (tabs[1][img] << 8) | (tabs[2][img] << 16)
        te_ref[img] = tabs[3][img]


def _tables(comb, eye):
    return pl.pallas_call(
        _tables_body,
        out_shape=[
            jax.ShapeDtypeStruct((4, 3, 256), I32),
            jax.ShapeDtypeStruct((4, 3, 256), I32),
        ],
    )(comb, eye)


@functools.cache
def _gather_sc_kernel():
    mesh = plsc.VectorSubcoreMesh(core_axis_name="c", subcore_axis_name="s")
    return pl.kernel(
        _gather_sc_body,
        out_type=jax.ShapeDtypeStruct((4 * 3 * NPIX,), I32),
        mesh=mesh,
        scratch_types=[
            pltpu.VMEM((2 * 4 * 3 * 256,), I32),
            pltpu.VMEM((CHUNK,), I32),
            pltpu.VMEM((CHUNK,), I32),
            pltpu.VMEM((CHUNK,), I32),
            pltpu.VMEM((CHUNK,), I32),
            pltpu.VMEM((CHUNK,), I32),
        ],
        compiler_params=pltpu.CompilerParams(needs_layout_passes=False),
    )


def _gather_sc_body(tab_hbm, p1_hbm, p2_hbm, out_hbm, tab, buf1, buf2, ob0, ob1, ob2):
    wid = lax.axis_index("s") * NC + lax.axis_index("c")
    base = wid * CHUNK
    obufs = [ob0, ob1, ob2]
    pltpu.sync_copy(tab_hbm, tab)
    for img in range(4):
        pltpu.sync_copy(p1_hbm.at[pl.ds(img * NPIX + base, CHUNK)], buf1)
        pltpu.sync_copy(p2_hbm.at[pl.ds(img * NPIX + base, CHUNK)], buf2)

        @pl.loop(0, CHUNK // 16, unroll=4)
        def _(i):
            w1 = buf1[pl.ds(i * 16, 16)]
            w2 = buf2[pl.ds(i * 16, 16)]
            for ch in range(3):
                vc = (w1 >> (8 * ch)) & 255
                vec = (w2 >> (8 * ch)) & 255
                g = plsc.load_gather(tab, [vc + (img * 3 + ch) * 256])
                ge = plsc.load_gather(tab, [vec + (3072 + (img * 3 + ch) * 256)])
                obufs[ch][pl.ds(i * 16, 16)] = g | (ge << 24)

        for ch in range(3):
            pltpu.sync_copy(obufs[ch], out_hbm.at[pl.ds((img * 3 + ch) * NPIX + base, CHUNK)])


def _compose_body(src_ref, tgt_ref, ms_ref, be_ref, mt_ref, lms_s_ref, lms_r_ref, out_ref):
    m_skin = ms_ref[0, 1]
    m_neck = ms_ref[0, 4]
    m_lip = ms_ref[0, 0]
    m_eyec = jnp.clip(ms_ref[0, 2] + ms_ref[0, 3], 0.0, 1.0)
    bew = be_ref[0]

    def mean_delta(lo, hi):
        n = float(hi - lo)

        def acc(j, c):
            return (c[0] + lms_s_ref[0, j, 0], c[1] + lms_s_ref[0, j, 1],
                    c[2] + lms_r_ref[0, j, 0], c[3] + lms_r_ref[0, j, 1])
        s0, s1, r0, r1 = lax.fori_loop(lo, hi, acc, (0.0, 0.0, 0.0, 0.0))
        d0 = (s0 / n - r0 / n + 1024.5).astype(I32) - 1024
        d1 = (s1 / n - r1 / n + 1024.5).astype(I32) - 1024
        return jnp.mod(d0, H), jnp.mod(d1, W)

    deltas = [mean_delta(48, 68), mean_delta(0, 68), mean_delta(36, 48)]
    regions = [(0.1, m_lip), (0.3, m_skin), (0.8, m_eyec)]

    for c in range(3):
        acc = src_ref[0, c]
        wc = mt_ref[0, c]
        for m, mk in ((0, m_skin), (1, m_neck), (2, m_lip)):
            t = ((wc >> (8 * m)) & 255).astype(F32) / 255.0 * 2.0 - 1.0
            acc = (1.0 - mk) * acc + mk * t
        t = ((wc >> 24) & 255).astype(F32) / 255.0 * 2.0 - 1.0
        acc = (1.0 - bew) * acc + bew * t
        tgt_c = tgt_ref[0, c]
        for (alpha, mk), (d0, d1) in zip(regions, deltas):
            rolled = pltpu.roll(pltpu.roll(tgt_c, d0, 0), d1, 1)
            wgt = alpha * mk
            acc = (1.0 - wgt) * acc + wgt * rolled
        out_ref[0, c] = acc


def _compose(sources, targets, mask_srcs, be, matched, lms_srcs, lms_tars):
    return pl.pallas_call(
        _compose_body,
        grid=(4,),
        in_specs=[
            pl.BlockSpec((1, 3, H, W), lambda i: (i, 0, 0, 0)),
            pl.BlockSpec((1, 3, H, W), lambda i: (i, 0, 0, 0)),
            pl.BlockSpec((1, 5, H, W), lambda i: (i, 0, 0, 0)),
            pl.BlockSpec((1, H, W), lambda i: (i, 0, 0)),
            pl.BlockSpec((1, 3, H, W), lambda i: (i, 0, 0, 0)),
            pl.BlockSpec((1, 68, 2), lambda i: (i, 0, 0), memory_space=pltpu.SMEM),
            pl.BlockSpec((1, 68, 2), lambda i: (i, 0, 0), memory_space=pltpu.SMEM),
        ],
        out_specs=pl.BlockSpec((1, 3, H, W), lambda i: (i, 0, 0, 0)),
        out_shape=jax.ShapeDtypeStruct((4, 3, H, W), F32),
    )(sources, targets, mask_srcs, be, matched, lms_srcs, lms_tars)


def kernel(sources, targets, mask_srcs, mask_tars, lms_srcs, lms_tars):
    hk, p1, p2, be = _prep(sources, targets, mask_srcs, mask_tars)
    p1f = p1.reshape(4 * NPIX)
    p2f = p2.reshape(4 * NPIX)
    histraw = _hist_sc_kernel()(hk.reshape(-1)).reshape(NW, HIST_W)
    comb = histraw[:, :NCOMB].reshape(NW, 24, 2048)
    eye = histraw[:, NCOMB:].reshape(NW, 24, 512)
    tb, te = _tables(comb, eye)
    tabflat = jnp.concatenate([tb.reshape(-1), te.reshape(-1)])
    matched = _gather_sc_kernel()(tabflat, p1f, p2f)
    return _compose(sources, targets, mask_srcs, be,
                    matched.reshape(4, 3, H, W), lms_srcs, lms_tars)


# speedup vs baseline: 1.1913x; 1.1913x over previous
import functools

import jax
import jax.numpy as jnp
from jax import lax
from jax.experimental import pallas as pl
from jax.experimental.pallas import tpu as pltpu
import jax.experimental.pallas.tpu_sc as plsc

H = W = 512
NPIX = H * W
NC, NS = 2, 16
NW = NC * NS
CHUNK = NPIX // NW
F32 = jnp.float32
I32 = jnp.int32


def _shz(x, s, axis):
    n = x.shape[axis]
    if s == 0:
        return x
    if axis == 0:
        if s > 0:
            return jnp.concatenate([x[s:, :], jnp.zeros((s, x.shape[1]), x.dtype)], 0)
        return jnp.concatenate([jnp.zeros((-s, x.shape[1]), x.dtype), x[: n + s, :]], 0)
    if s > 0:
        return jnp.concatenate([x[:, s:], jnp.zeros((x.shape[0], s), x.dtype)], 1)
    return jnp.concatenate([jnp.zeros((x.shape[0], -s), x.dtype), x[:, : n + s]], 1)


def _maxpool25_axis(x, axis):
    f2 = jnp.maximum(x, _shz(x, 1, axis))
    f4 = jnp.maximum(f2, _shz(f2, 2, axis))
    f8 = jnp.maximum(f4, _shz(f4, 4, axis))
    r13 = jnp.maximum(f8, _shz(f8, 5, axis))
    b2 = jnp.maximum(x, _shz(x, -1, axis))
    b4 = jnp.maximum(b2, _shz(b2, -2, axis))
    b8 = jnp.maximum(b4, _shz(b4, -4, axis))
    l13 = jnp.maximum(b8, _shz(b8, -5, axis))
    return jnp.maximum(r13, l13)


def _expand25(m):
    return _maxpool25_axis(_maxpool25_axis(m, 0), 1)


def _blur5_valid(m):
    r = m
    r = _shz(m, -2, 0) + _shz(m, -1, 0) + m + _shz(m, 1, 0) + _shz(m, 2, 0)
    r = _shz(r, -2, 1) + _shz(r, -1, 1) + r + _shz(r, 1, 1) + _shz(r, 2, 1)
    return (r / 25.0) * m


def _prep_body(src_ref, tgt_ref, ms_ref, mr_ref, hk_ref, p1_ref, p2_ref, be_ref):
    i2 = pl.program_id(0) * 2
    for side in range(2):
        img_ref = src_ref if side == 0 else tgt_ref
        msk_ref = ms_ref if side == 0 else mr_ref
        img = jnp.clip((img_ref[0] + 1.0) / 2.0, 0.0, 1.0) * 255.0
        m0 = msk_ref[0, 0]
        m1 = msk_ref[0, 1]
        m4 = msk_ref[0, 4]
        me = _expand25(msk_ref[0, 2] + msk_ref[0, 3]) * m1
        v = [jnp.clip(img[c], 0.0, 255.0).astype(I32) for c in range(3)]
        ve = [jnp.clip(img[c] * me, 0.0, 255.0).astype(I32) for c in range(3)]
        bits3 = ((m1 > 0).astype(I32) | ((m4 > 0).astype(I32) << 1)
                 | ((m0 > 0).astype(I32) << 2)) << 8
        beye = (me > 0).astype(I32) << 8
        isb = (i2 + side) * 3
        for ch in range(3):
            kc = (bits3 | v[ch]) + (isb + ch) * 2048
            ke = (beye | ve[ch]) + (NCOMB + (isb + ch) * 512)
            hk_ref[0, side, ch] = kc[:256, :] | (kc[256:, :] << 16)
            hk_ref[0, side, 3 + ch] = ke[:256, :] | (ke[256:, :] << 16)
        if side == 0:
            p1_ref[0] = v[0] | (v[1] << 8) | (v[2] << 16)
            p2_ref[0] = ve[0] | (ve[1] << 8) | (ve[2] << 16)
            be_ref[0] = _blur5_valid(me)


def _prep(sources, targets, mask_srcs, mask_tars):
    return pl.pallas_call(
        _prep_body,
        grid=(4,),
        in_specs=[
            pl.BlockSpec((1, 3, H, W), lambda i: (i, 0, 0, 0)),
            pl.BlockSpec((1, 3, H, W), lambda i: (i, 0, 0, 0)),
            pl.BlockSpec((1, 5, H, W), lambda i: (i, 0, 0, 0)),
            pl.BlockSpec((1, 5, H, W), lambda i: (i, 0, 0, 0)),
        ],
        out_specs=[
            pl.BlockSpec((1, 2, 6, H // 2, W), lambda i: (i, 0, 0, 0, 0)),
            pl.BlockSpec((1, H, W), lambda i: (i, 0, 0)),
            pl.BlockSpec((1, H, W), lambda i: (i, 0, 0)),
            pl.BlockSpec((1, H, W), lambda i: (i, 0, 0)),
        ],
        out_shape=[
            jax.ShapeDtypeStruct((4, 2, 6, H // 2, W), I32),
            jax.ShapeDtypeStruct((4, H, W), I32),
            jax.ShapeDtypeStruct((4, H, W), I32),
            jax.ShapeDtypeStruct((4, H, W), F32),
        ],
    )(sources, targets, mask_srcs, mask_tars)


NCOMB = 4 * 2 * 3 * 8 * 256
NEYE = 4 * 2 * 3 * 2 * 256
HIST_W = NCOMB + NEYE


NKW = NPIX // 2
KCH = NKW // NW
SIDE_W = 6 * KCH


@functools.cache
def _hist_sc_kernel():
    mesh = plsc.VectorSubcoreMesh(core_axis_name="c", subcore_axis_name="s")
    return pl.kernel(
        _hist_sc_body,
        out_type=jax.ShapeDtypeStruct((NW * HIST_W,), F32),
        mesh=mesh,
        scratch_types=[
            pltpu.VMEM((SIDE_W,), I32),
            pltpu.VMEM((SIDE_W,), I32),
            pltpu.VMEM((HIST_W,), F32),
            pltpu.SemaphoreType.DMA,
        ],
        compiler_params=pltpu.CompilerParams(needs_layout_passes=False),
    )


def _hist_sc_body(hk_hbm, out_hbm, bufa, bufb, hist, sem):
    wid = lax.axis_index("s") * NC + lax.axis_index("c")
    zeros = jnp.zeros((16,), F32)
    ones = jnp.full((16,), 1.0, F32)
    bufs = [bufa, bufb]

    def issue(k, buf):
        descs = []
        for kt in range(6):
            off = (k * 6 + kt) * NKW + wid * KCH
            descs.append(pltpu.async_copy(
                hk_hbm.at[pl.ds(off, KCH)], buf.at[pl.ds(kt * KCH, KCH)], sem))
        return descs

    descs = issue(0, bufs[0])

    @plsc.parallel_loop(0, HIST_W // 16, unroll=8)
    def _(i):
        hist[pl.ds(i * 16, 16)] = zeros

    for k in range(8):
        buf = bufs[k % 2]
        for d in descs:
            d.wait()
        if k < 7:
            descs = issue(k + 1, bufs[(k + 1) % 2])

        @plsc.parallel_loop(0, SIDE_W // 16, unroll=8)
        def _(i):
            w = buf[pl.ds(i * 16, 16)]
            plsc.addupdate_scatter(hist, [w & 0xFFFF], ones)
            plsc.addupdate_scatter(hist, [lax.shift_right_logical(w, 16)], ones)

    pltpu.sync_copy(hist, out_hbm.at[pl.ds(wid * HIST_W, HIST_W)])


def _tables_body(comb_ref, eye_ref, tb_ref, te_ref):
    hc = jnp.sum(comb_ref[...], axis=0)
    he = jnp.sum(eye_ref[...], axis=0)[:, 256:512]
    combos = [hc[:, b * 256:(b + 1) * 256] for b in range(8)]
    hists = [
        combos[1] + combos[3] + combos[5] + combos[7],
        combos[2] + combos[3] + combos[6] + combos[7],
        combos[4] + combos[5] + combos[6] + combos[7],
        he,
    ]
    jj = lax.broadcasted_iota(I32, (256, 256), 0)
    ii = lax.broadcasted_iota(I32, (256, 256), 1)
    tri = (jj <= ii).astype(F32)
    tabs = []
    for h in hists:
        cs = lax.dot_general(h, tri, (((1,), (0,)), ((), ())),
                             precision=lax.Precision.HIGHEST,
                             preferred_element_type=F32)
        csn = cs / jnp.maximum(cs[:, 255:256], 1.0)
        tm = []
        for img in range(4):
            a = csn[img * 6: img * 6 + 3]
            b = csn[img * 6 + 3: img * 6 + 6]
            aa = jnp.broadcast_to(a[:, :, None], (3, 256, 256))
            bb = jnp.broadcast_to(b[:, None, :], (3, 256, 256))
            cnt = jnp.sum((bb < aa).astype(F32), axis=2)
            tm.append(jnp.clip(cnt, 0.0, 255.0).astype(I32))
        tabs.append(tm)
    for img in range(4):
        tb_ref[img] = tabs[0][img] | (tabs[1][img] << 8) | (tabs[2][img] << 16)
        te_ref[img] = tabs[3][img]


def _tables(comb, eye):
    return pl.pallas_call(
        _tables_body,
        out_shape=[
            jax.ShapeDtypeStruct((4, 3, 256), I32),
            jax.ShapeDtypeStruct((4, 3, 256), I32),
        ],
    )(comb, eye)


@functools.cache
def _gather_sc_kernel():
    mesh = plsc.VectorSubcoreMesh(core_axis_name="c", subcore_axis_name="s")
    return pl.kernel(
        _gather_sc_body,
        out_type=jax.ShapeDtypeStruct((4 * 3 * NPIX,), I32),
        mesh=mesh,
        scratch_types=[
            pltpu.VMEM((2 * 4 * 3 * 256,), I32),
            pltpu.VMEM((CHUNK,), I32),
            pltpu.VMEM((CHUNK,), I32),
            pltpu.VMEM((CHUNK,), I32),
            pltpu.VMEM((CHUNK,), I32),
            pltpu.VMEM((CHUNK,), I32),
        ],
        compiler_params=pltpu.CompilerParams(needs_layout_passes=False),
    )


def _gather_sc_body(tab_hbm, p1_hbm, p2_hbm, out_hbm, tab, buf1, buf2, ob0, ob1, ob2):
    wid = lax.axis_index("s") * NC + lax.axis_index("c")
    base = wid * CHUNK
    obufs = [ob0, ob1, ob2]
    pltpu.sync_copy(tab_hbm, tab)
    for img in range(4):
        pltpu.sync_copy(p1_hbm.at[pl.ds(img * NPIX + base, CHUNK)], buf1)
        pltpu.sync_copy(p2_hbm.at[pl.ds(img * NPIX + base, CHUNK)], buf2)

        @plsc.parallel_loop(0, CHUNK // 16, unroll=4)
        def _(i):
            w1 = buf1[pl.ds(i * 16, 16)]
            w2 = buf2[pl.ds(i * 16, 16)]
            for ch in range(3):
                vc = (w1 >> (8 * ch)) & 255
                vec = (w2 >> (8 * ch)) & 255
                g = plsc.load_gather(tab, [vc + (img * 3 + ch) * 256])
                ge = plsc.load_gather(tab, [vec + (3072 + (img * 3 + ch) * 256)])
                obufs[ch][pl.ds(i * 16, 16)] = g | (ge << 24)

        for ch in range(3):
            pltpu.sync_copy(obufs[ch], out_hbm.at[pl.ds((img * 3 + ch) * NPIX + base, CHUNK)])


def _compose_body(src_ref, tgt_ref, ms_ref, be_ref, mt_ref, lms_s_ref, lms_r_ref, out_ref):
    m_skin = ms_ref[0, 1]
    m_neck = ms_ref[0, 4]
    m_lip = ms_ref[0, 0]
    m_eyec = jnp.clip(ms_ref[0, 2] + ms_ref[0, 3], 0.0, 1.0)
    bew = be_ref[0]

    def mean_delta(lo, hi):
        n = float(hi - lo)

        def acc(j, c):
            return (c[0] + lms_s_ref[0, j, 0], c[1] + lms_s_ref[0, j, 1],
                    c[2] + lms_r_ref[0, j, 0], c[3] + lms_r_ref[0, j, 1])
        s0, s1, r0, r1 = lax.fori_loop(lo, hi, acc, (0.0, 0.0, 0.0, 0.0))
        d0 = (s0 / n - r0 / n + 1024.5).astype(I32) - 1024
        d1 = (s1 / n - r1 / n + 1024.5).astype(I32) - 1024
        return jnp.mod(d0, H), jnp.mod(d1, W)

    deltas = [mean_delta(48, 68), mean_delta(0, 68), mean_delta(36, 48)]
    regions = [(0.1, m_lip), (0.3, m_skin), (0.8, m_eyec)]

    for c in range(3):
        acc = src_ref[0, c]
        wc = mt_ref[0, c]
        for m, mk in ((0, m_skin), (1, m_neck), (2, m_lip)):
            t = ((wc >> (8 * m)) & 255).astype(F32) / 255.0 * 2.0 - 1.0
            acc = (1.0 - mk) * acc + mk * t
        t = ((wc >> 24) & 255).astype(F32) / 255.0 * 2.0 - 1.0
        acc = (1.0 - bew) * acc + bew * t
        tgt_c = tgt_ref[0, c]
        for (alpha, mk), (d0, d1) in zip(regions, deltas):
            rolled = pltpu.roll(pltpu.roll(tgt_c, d0, 0), d1, 1)
            wgt = alpha * mk
            acc = (1.0 - wgt) * acc + wgt * rolled
        out_ref[0, c] = acc


def _compose(sources, targets, mask_srcs, be, matched, lms_srcs, lms_tars):
    return pl.pallas_call(
        _compose_body,
        grid=(4,),
        in_specs=[
            pl.BlockSpec((1, 3, H, W), lambda i: (i, 0, 0, 0)),
            pl.BlockSpec((1, 3, H, W), lambda i: (i, 0, 0, 0)),
            pl.BlockSpec((1, 5, H, W), lambda i: (i, 0, 0, 0)),
            pl.BlockSpec((1, H, W), lambda i: (i, 0, 0)),
            pl.BlockSpec((1, 3, H, W), lambda i: (i, 0, 0, 0)),
            pl.BlockSpec((1, 68, 2), lambda i: (i, 0, 0), memory_space=pltpu.SMEM),
            pl.BlockSpec((1, 68, 2), lambda i: (i, 0, 0), memory_space=pltpu.SMEM),
        ],
        out_specs=pl.BlockSpec((1, 3, H, W), lambda i: (i, 0, 0, 0)),
        out_shape=jax.ShapeDtypeStruct((4, 3, H, W), F32),
    )(sources, targets, mask_srcs, be, matched, lms_srcs, lms_tars)


def kernel(sources, targets, mask_srcs, mask_tars, lms_srcs, lms_tars):
    hk, p1, p2, be = _prep(sources, targets, mask_srcs, mask_tars)
    p1f = p1.reshape(4 * NPIX)
    p2f = p2.reshape(4 * NPIX)
    histraw = _hist_sc_kernel()(hk.reshape(-1)).reshape(NW, HIST_W)
    comb = histraw[:, :NCOMB].reshape(NW, 24, 2048)
    eye = histraw[:, NCOMB:].reshape(NW, 24, 512)
    tb, te = _tables(comb, eye)
    tabflat = jnp.concatenate([tb.reshape(-1), te.reshape(-1)])
    matched = _gather_sc_kernel()(tabflat, p1f, p2f)
    return _compose(sources, targets, mask_srcs, be,
                    matched.reshape(4, 3, H, W), lms_srcs, lms_tars)


# speedup vs baseline: 1.8010x; 1.5117x over previous
import functools

import jax
import jax.numpy as jnp
from jax import lax
from jax.experimental import pallas as pl
from jax.experimental.pallas import tpu as pltpu
import jax.experimental.pallas.tpu_sc as plsc

H = W = 512
NPIX = H * W
NC, NS = 2, 16
NW = NC * NS
CHUNK = NPIX // NW
F32 = jnp.float32
I32 = jnp.int32


def _shz(x, s, axis):
    n = x.shape[axis]
    if s == 0:
        return x
    if axis == 0:
        if s > 0:
            return jnp.concatenate([x[s:, :], jnp.zeros((s, x.shape[1]), x.dtype)], 0)
        return jnp.concatenate([jnp.zeros((-s, x.shape[1]), x.dtype), x[: n + s, :]], 0)
    if s > 0:
        return jnp.concatenate([x[:, s:], jnp.zeros((x.shape[0], s), x.dtype)], 1)
    return jnp.concatenate([jnp.zeros((x.shape[0], -s), x.dtype), x[:, : n + s]], 1)


def _maxpool25_axis(x, axis):
    f2 = jnp.maximum(x, _shz(x, 1, axis))
    f4 = jnp.maximum(f2, _shz(f2, 2, axis))
    f8 = jnp.maximum(f4, _shz(f4, 4, axis))
    r13 = jnp.maximum(f8, _shz(f8, 5, axis))
    b2 = jnp.maximum(x, _shz(x, -1, axis))
    b4 = jnp.maximum(b2, _shz(b2, -2, axis))
    b8 = jnp.maximum(b4, _shz(b4, -4, axis))
    l13 = jnp.maximum(b8, _shz(b8, -5, axis))
    return jnp.maximum(r13, l13)


def _expand25(m):
    return _maxpool25_axis(_maxpool25_axis(m, 0), 1)


def _blur5_valid(m):
    r = m
    r = _shz(m, -2, 0) + _shz(m, -1, 0) + m + _shz(m, 1, 0) + _shz(m, 2, 0)
    r = _shz(r, -2, 1) + _shz(r, -1, 1) + r + _shz(r, 1, 1) + _shz(r, 2, 1)
    return (r / 25.0) * m


def _prep_body(src_ref, tgt_ref, ms_ref, mr_ref, hk_ref, p1_ref, p2_ref, be_ref):
    i2 = pl.program_id(0) * 2
    for side in range(2):
        img_ref = src_ref if side == 0 else tgt_ref
        msk_ref = ms_ref if side == 0 else mr_ref
        img = jnp.clip((img_ref[0] + 1.0) / 2.0, 0.0, 1.0) * 255.0
        m0 = msk_ref[0, 0]
        m1 = msk_ref[0, 1]
        m4 = msk_ref[0, 4]
        me = _expand25(msk_ref[0, 2] + msk_ref[0, 3]) * m1
        v = [jnp.clip(img[c], 0.0, 255.0).astype(I32) for c in range(3)]
        ve = [jnp.clip(img[c] * me, 0.0, 255.0).astype(I32) for c in range(3)]
        bits3 = ((m1 > 0).astype(I32) | ((m4 > 0).astype(I32) << 1)
                 | ((m0 > 0).astype(I32) << 2)) << 8
        beye = (me > 0).astype(I32) << 8
        isb = (i2 + side) * 3
        for ch in range(3):
            kc = (bits3 | v[ch]) + (isb + ch) * 2048
            ke = (beye | jnp.where(me > 0, ve[ch], v[ch])) \
                + (NCOMB + (isb + ch) * 512)
            hk_ref[0, side, ch] = kc[:256, :] | (kc[256:, :] << 16)
            hk_ref[0, side, 3 + ch] = ke[:256, :] | (ke[256:, :] << 16)
        if side == 0:
            p1_ref[0] = v[0] | (v[1] << 8) | (v[2] << 16)
            p2_ref[0] = ve[0] | (ve[1] << 8) | (ve[2] << 16)
            be_ref[0] = _blur5_valid(me)


def _prep(sources, targets, mask_srcs, mask_tars):
    return pl.pallas_call(
        _prep_body,
        grid=(4,),
        in_specs=[
            pl.BlockSpec((1, 3, H, W), lambda i: (i, 0, 0, 0)),
            pl.BlockSpec((1, 3, H, W), lambda i: (i, 0, 0, 0)),
            pl.BlockSpec((1, 5, H, W), lambda i: (i, 0, 0, 0)),
            pl.BlockSpec((1, 5, H, W), lambda i: (i, 0, 0, 0)),
        ],
        out_specs=[
            pl.BlockSpec((1, 2, 6, H // 2, W), lambda i: (i, 0, 0, 0, 0)),
            pl.BlockSpec((1, H, W), lambda i: (i, 0, 0)),
            pl.BlockSpec((1, H, W), lambda i: (i, 0, 0)),
            pl.BlockSpec((1, H, W), lambda i: (i, 0, 0)),
        ],
        out_shape=[
            jax.ShapeDtypeStruct((4, 2, 6, H // 2, W), I32),
            jax.ShapeDtypeStruct((4, H, W), I32),
            jax.ShapeDtypeStruct((4, H, W), I32),
            jax.ShapeDtypeStruct((4, H, W), F32),
        ],
    )(sources, targets, mask_srcs, mask_tars)


NCOMB = 4 * 2 * 3 * 8 * 256
NEYE = 4 * 2 * 3 * 2 * 256
HIST_W = NCOMB + NEYE


NKW = NPIX // 2
KCH = NKW // NW
SIDE_W = 6 * KCH


@functools.cache
def _hist_sc_kernel():
    mesh = plsc.VectorSubcoreMesh(core_axis_name="c", subcore_axis_name="s")
    return pl.kernel(
        _hist_sc_body,
        out_type=jax.ShapeDtypeStruct((NW * HIST_W,), F32),
        mesh=mesh,
        scratch_types=[
            pltpu.VMEM((SIDE_W,), I32),
            pltpu.VMEM((SIDE_W,), I32),
            pltpu.VMEM((HIST_W,), F32),
            pltpu.SemaphoreType.DMA,
        ],
        compiler_params=pltpu.CompilerParams(needs_layout_passes=False),
    )


def _hist_sc_body(hk_hbm, out_hbm, bufa, bufb, hist, sem):
    wid = lax.axis_index("s") * NC + lax.axis_index("c")
    zeros = jnp.zeros((16,), F32)
    ones = jnp.full((16,), 1.0, F32)
    bufs = [bufa, bufb]

    def issue(k, buf):
        descs = []
        for kt in range(6):
            off = (k * 6 + kt) * NKW + wid * KCH
            descs.append(pltpu.async_copy(
                hk_hbm.at[pl.ds(off, KCH)], buf.at[pl.ds(kt * KCH, KCH)], sem))
        return descs

    descs = issue(0, bufs[0])

    @plsc.parallel_loop(0, HIST_W // 16, unroll=8)
    def _(i):
        hist[pl.ds(i * 16, 16)] = zeros

    for k in range(8):
        buf = bufs[k % 2]
        for d in descs:
            d.wait()
        if k < 7:
            descs = issue(k + 1, bufs[(k + 1) % 2])

        @plsc.parallel_loop(0, SIDE_W // 16, unroll=8)
        def _(i):
            w = buf[pl.ds(i * 16, 16)]
            plsc.addupdate_scatter(hist, [w & 0xFFFF], ones)
            plsc.addupdate_scatter(hist, [lax.shift_right_logical(w, 16)], ones)

    pltpu.sync_copy(hist, out_hbm.at[pl.ds(wid * HIST_W, HIST_W)])


def _tables_body(comb_ref, eye_ref, tb_ref, te_ref):
    hc = jnp.sum(comb_ref[...], axis=0)
    he = jnp.sum(eye_ref[...], axis=0)[:, 256:512]
    combos = [hc[:, b * 256:(b + 1) * 256] for b in range(8)]
    hists = [
        combos[1] + combos[3] + combos[5] + combos[7],
        combos[2] + combos[3] + combos[6] + combos[7],
        combos[4] + combos[5] + combos[6] + combos[7],
        he,
    ]
    jj = lax.broadcasted_iota(I32, (256, 256), 0)
    ii = lax.broadcasted_iota(I32, (256, 256), 1)
    tri = (jj <= ii).astype(F32)
    tabs = []
    for h in hists:
        cs = lax.dot_general(h, tri, (((1,), (0,)), ((), ())),
                             precision=lax.Precision.HIGHEST,
                             preferred_element_type=F32)
        csn = cs / jnp.maximum(cs[:, 255:256], 1.0)
        tm = []
        for img in range(4):
            a = csn[img * 6: img * 6 + 3]
            b = csn[img * 6 + 3: img * 6 + 6]
            aa = jnp.broadcast_to(a[:, :, None], (3, 256, 256))
            bb = jnp.broadcast_to(b[:, None, :], (3, 256, 256))
            cnt = jnp.sum((bb < aa).astype(F32), axis=2)
            tm.append(jnp.clip(cnt, 0.0, 255.0).astype(I32))
        tabs.append(tm)
    for img in range(4):
        tb_ref[img] = tabs[0][img] | (tabs[1][img] << 8) | (tabs[2][img] << 16)
        te_ref[img] = tabs[3][img]


def _tables(comb, eye):
    return pl.pallas_call(
        _tables_body,
        out_shape=[
            jax.ShapeDtypeStruct((4, 3, 256), I32),
            jax.ShapeDtypeStruct((4, 3, 256), I32),
        ],
    )(comb, eye)


@functools.cache
def _gather_sc_kernel():
    mesh = plsc.VectorSubcoreMesh(core_axis_name="c", subcore_axis_name="s")
    return pl.kernel(
        _gather_sc_body,
        out_type=jax.ShapeDtypeStruct((4 * 3 * NPIX,), I32),
        mesh=mesh,
        scratch_types=[
            pltpu.VMEM((2 * 4 * 3 * 256,), I32),
            pltpu.VMEM((CHUNK,), I32),
            pltpu.VMEM((CHUNK,), I32),
            pltpu.VMEM((CHUNK,), I32),
            pltpu.VMEM((CHUNK,), I32),
            pltpu.VMEM((CHUNK,), I32),
        ],
        compiler_params=pltpu.CompilerParams(needs_layout_passes=False),
    )


def _gather_sc_body(tab_hbm, p1_hbm, p2_hbm, out_hbm, tab, buf1, buf2, ob0, ob1, ob2):
    wid = lax.axis_index("s") * NC + lax.axis_index("c")
    base = wid * CHUNK
    obufs = [ob0, ob1, ob2]
    pltpu.sync_copy(tab_hbm, tab)
    for img in range(4):
        pltpu.sync_copy(p1_hbm.at[pl.ds(img * NPIX + base, CHUNK)], buf1)
        pltpu.sync_copy(p2_hbm.at[pl.ds(img * NPIX + base, CHUNK)], buf2)

        @plsc.parallel_loop(0, CHUNK // 16, unroll=4)
        def _(i):
            w1 = buf1[pl.ds(i * 16, 16)]
            w2 = buf2[pl.ds(i * 16, 16)]
            for ch in range(3):
                vc = (w1 >> (8 * ch)) & 255
                vec = (w2 >> (8 * ch)) & 255
                g = plsc.load_gather(tab, [vc + (img * 3 + ch) * 256])
                ge = plsc.load_gather(tab, [vec + (3072 + (img * 3 + ch) * 256)])
                obufs[ch][pl.ds(i * 16, 16)] = g | (ge << 24)

        for ch in range(3):
            pltpu.sync_copy(obufs[ch], out_hbm.at[pl.ds((img * 3 + ch) * NPIX + base, CHUNK)])


def _compose_body(src_ref, tgt_ref, ms_ref, be_ref, mt_ref, lms_s_ref, lms_r_ref, out_ref):
    m_skin = ms_ref[0, 1]
    m_neck = ms_ref[0, 4]
    m_lip = ms_ref[0, 0]
    m_eyec = jnp.clip(ms_ref[0, 2] + ms_ref[0, 3], 0.0, 1.0)
    bew = be_ref[0]

    def mean_delta(lo, hi):
        n = float(hi - lo)

        def acc(j, c):
            return (c[0] + lms_s_ref[0, j, 0], c[1] + lms_s_ref[0, j, 1],
                    c[2] + lms_r_ref[0, j, 0], c[3] + lms_r_ref[0, j, 1])
        s0, s1, r0, r1 = lax.fori_loop(lo, hi, acc, (0.0, 0.0, 0.0, 0.0))
        d0 = (s0 / n - r0 / n + 1024.5).astype(I32) - 1024
        d1 = (s1 / n - r1 / n + 1024.5).astype(I32) - 1024
        return jnp.mod(d0, H), jnp.mod(d1, W)

    deltas = [mean_delta(48, 68), mean_delta(0, 68), mean_delta(36, 48)]
    regions = [(0.1, m_lip), (0.3, m_skin), (0.8, m_eyec)]

    for c in range(3):
        acc = src_ref[0, c]
        wc = mt_ref[0, c]
        for m, mk in ((0, m_skin), (1, m_neck), (2, m_lip)):
            t = ((wc >> (8 * m)) & 255).astype(F32) / 255.0 * 2.0 - 1.0
            acc = (1.0 - mk) * acc + mk * t
        t = ((wc >> 24) & 255).astype(F32) / 255.0 * 2.0 - 1.0
        acc = (1.0 - bew) * acc + bew * t
        tgt_c = tgt_ref[0, c]
        for (alpha, mk), (d0, d1) in zip(regions, deltas):
            rolled = pltpu.roll(pltpu.roll(tgt_c, d0, 0), d1, 1)
            wgt = alpha * mk
            acc = (1.0 - wgt) * acc + wgt * rolled
        out_ref[0, c] = acc


def _compose(sources, targets, mask_srcs, be, matched, lms_srcs, lms_tars):
    return pl.pallas_call(
        _compose_body,
        grid=(4,),
        in_specs=[
            pl.BlockSpec((1, 3, H, W), lambda i: (i, 0, 0, 0)),
            pl.BlockSpec((1, 3, H, W), lambda i: (i, 0, 0, 0)),
            pl.BlockSpec((1, 5, H, W), lambda i: (i, 0, 0, 0)),
            pl.BlockSpec((1, H, W), lambda i: (i, 0, 0)),
            pl.BlockSpec((1, 3, H, W), lambda i: (i, 0, 0, 0)),
            pl.BlockSpec((1, 68, 2), lambda i: (i, 0, 0), memory_space=pltpu.SMEM),
            pl.BlockSpec((1, 68, 2), lambda i: (i, 0, 0), memory_space=pltpu.SMEM),
        ],
        out_specs=pl.BlockSpec((1, 3, H, W), lambda i: (i, 0, 0, 0)),
        out_shape=jax.ShapeDtypeStruct((4, 3, H, W), F32),
    )(sources, targets, mask_srcs, be, matched, lms_srcs, lms_tars)


def kernel(sources, targets, mask_srcs, mask_tars, lms_srcs, lms_tars):
    hk, p1, p2, be = _prep(sources, targets, mask_srcs, mask_tars)
    p1f = p1.reshape(4 * NPIX)
    p2f = p2.reshape(4 * NPIX)
    histraw = _hist_sc_kernel()(hk.reshape(-1)).reshape(NW, HIST_W)
    comb = histraw[:, :NCOMB].reshape(NW, 24, 2048)
    eye = histraw[:, NCOMB:].reshape(NW, 24, 512)
    tb, te = _tables(comb, eye)
    tabflat = jnp.concatenate([tb.reshape(-1), te.reshape(-1)])
    matched = _gather_sc_kernel()(tabflat, p1f, p2f)
    return _compose(sources, targets, mask_srcs, be,
                    matched.reshape(4, 3, H, W), lms_srcs, lms_tars)


# speedup vs baseline: 1.9203x; 1.0663x over previous
import functools

import jax
import jax.numpy as jnp
from jax import lax
from jax.experimental import pallas as pl
from jax.experimental.pallas import tpu as pltpu
import jax.experimental.pallas.tpu_sc as plsc

H = W = 512
NPIX = H * W
NC, NS = 2, 16
NW = NC * NS
CHUNK = NPIX // NW
F32 = jnp.float32
I32 = jnp.int32


def _shz(x, s, axis):
    n = x.shape[axis]
    if s == 0:
        return x
    if axis == 0:
        if s > 0:
            return jnp.concatenate([x[s:, :], jnp.zeros((s, x.shape[1]), x.dtype)], 0)
        return jnp.concatenate([jnp.zeros((-s, x.shape[1]), x.dtype), x[: n + s, :]], 0)
    if s > 0:
        return jnp.concatenate([x[:, s:], jnp.zeros((x.shape[0], s), x.dtype)], 1)
    return jnp.concatenate([jnp.zeros((x.shape[0], -s), x.dtype), x[:, : n + s]], 1)


def _maxpool25_axis(x, axis):
    f2 = jnp.maximum(x, _shz(x, 1, axis))
    f4 = jnp.maximum(f2, _shz(f2, 2, axis))
    f8 = jnp.maximum(f4, _shz(f4, 4, axis))
    r13 = jnp.maximum(f8, _shz(f8, 5, axis))
    b2 = jnp.maximum(x, _shz(x, -1, axis))
    b4 = jnp.maximum(b2, _shz(b2, -2, axis))
    b8 = jnp.maximum(b4, _shz(b4, -4, axis))
    l13 = jnp.maximum(b8, _shz(b8, -5, axis))
    return jnp.maximum(r13, l13)


def _expand25(m):
    return _maxpool25_axis(_maxpool25_axis(m, 0), 1)


def _blur5_valid(m):
    r = m
    r = _shz(m, -2, 0) + _shz(m, -1, 0) + m + _shz(m, 1, 0) + _shz(m, 2, 0)
    r = _shz(r, -2, 1) + _shz(r, -1, 1) + r + _shz(r, 1, 1) + _shz(r, 2, 1)
    return (r / 25.0) * m


def _prep_body(src_ref, tgt_ref, ms_ref, mr_ref, hk_ref, p1_ref, p2_ref, be_ref):
    i2 = pl.program_id(0) * 2
    for side in range(2):
        img_ref = src_ref if side == 0 else tgt_ref
        msk_ref = ms_ref if side == 0 else mr_ref
        img = jnp.clip((img_ref[0] + 1.0) / 2.0, 0.0, 1.0) * 255.0
        m0 = msk_ref[0, 0]
        m1 = msk_ref[0, 1]
        m4 = msk_ref[0, 4]
        me = _expand25(msk_ref[0, 2] + msk_ref[0, 3]) * m1
        v = [jnp.clip(img[c], 0.0, 255.0).astype(I32) for c in range(3)]
        ve = [jnp.clip(img[c] * me, 0.0, 255.0).astype(I32) for c in range(3)]
        bits3 = ((m1 > 0).astype(I32) | ((m4 > 0).astype(I32) << 1)
                 | ((m0 > 0).astype(I32) << 2)) << 8
        beye = (me > 0).astype(I32) << 8
        isb = (i2 + side) * 3
        for ch in range(3):
            kc = (bits3 | v[ch]) + (isb + ch) * 2048
            ke = (beye | jnp.where(me > 0, ve[ch], v[ch])) \
                + (NCOMB + (isb + ch) * 512)
            hk_ref[0, side, ch] = kc[:256, :] | (kc[256:, :] << 16)
            hk_ref[0, side, 3 + ch] = ke[:256, :] | (ke[256:, :] << 16)
        if side == 0:
            p1_ref[0] = v[0] | (v[1] << 8) | (v[2] << 16)
            p2_ref[0] = ve[0] | (ve[1] << 8) | (ve[2] << 16)
            be_ref[0] = _blur5_valid(me)


def _prep(sources, targets, mask_srcs, mask_tars):
    return pl.pallas_call(
        _prep_body,
        grid=(4,),
        in_specs=[
            pl.BlockSpec((1, 3, H, W), lambda i: (i, 0, 0, 0)),
            pl.BlockSpec((1, 3, H, W), lambda i: (i, 0, 0, 0)),
            pl.BlockSpec((1, 5, H, W), lambda i: (i, 0, 0, 0)),
            pl.BlockSpec((1, 5, H, W), lambda i: (i, 0, 0, 0)),
        ],
        out_specs=[
            pl.BlockSpec((1, 2, 6, H // 2, W), lambda i: (i, 0, 0, 0, 0)),
            pl.BlockSpec((1, H, W), lambda i: (i, 0, 0)),
            pl.BlockSpec((1, H, W), lambda i: (i, 0, 0)),
            pl.BlockSpec((1, H, W), lambda i: (i, 0, 0)),
        ],
        out_shape=[
            jax.ShapeDtypeStruct((4, 2, 6, H // 2, W), I32),
            jax.ShapeDtypeStruct((4, H, W), I32),
            jax.ShapeDtypeStruct((4, H, W), I32),
            jax.ShapeDtypeStruct((4, H, W), F32),
        ],
    )(sources, targets, mask_srcs, mask_tars)


NCOMB = 4 * 2 * 3 * 8 * 256
NEYE = 4 * 2 * 3 * 2 * 256
HIST_W = NCOMB + NEYE


NKW = NPIX // 2
KCH = NKW // NW
SIDE_W = 6 * KCH


@functools.cache
def _hist_sc_kernel():
    mesh = plsc.VectorSubcoreMesh(core_axis_name="c", subcore_axis_name="s")
    return pl.kernel(
        _hist_sc_body,
        out_type=[
            jax.ShapeDtypeStruct((NW * NCOMB,), F32),
            jax.ShapeDtypeStruct((NW * NEYE,), F32),
        ],
        mesh=mesh,
        scratch_types=[
            pltpu.VMEM((SIDE_W,), I32),
            pltpu.VMEM((SIDE_W,), I32),
            pltpu.VMEM((HIST_W,), F32),
            pltpu.SemaphoreType.DMA,
        ],
        compiler_params=pltpu.CompilerParams(needs_layout_passes=False),
    )


def _hist_sc_body(hk_hbm, outc_hbm, oute_hbm, bufa, bufb, hist, sem):
    wid = lax.axis_index("s") * NC + lax.axis_index("c")
    zeros = jnp.zeros((16,), F32)
    ones = jnp.full((16,), 1.0, F32)
    bufs = [bufa, bufb]

    def issue(k, buf):
        descs = []
        for kt in range(6):
            off = (k * 6 + kt) * NKW + wid * KCH
            descs.append(pltpu.async_copy(
                hk_hbm.at[pl.ds(off, KCH)], buf.at[pl.ds(kt * KCH, KCH)], sem))
        return descs

    descs = issue(0, bufs[0])

    @plsc.parallel_loop(0, HIST_W // 16, unroll=8)
    def _(i):
        hist[pl.ds(i * 16, 16)] = zeros

    for k in range(8):
        buf = bufs[k % 2]
        for d in descs:
            d.wait()
        if k < 7:
            descs = issue(k + 1, bufs[(k + 1) % 2])

        @plsc.parallel_loop(0, SIDE_W // 16, unroll=8)
        def _(i):
            w = buf[pl.ds(i * 16, 16)]
            plsc.addupdate_scatter(hist, [w & 0xFFFF], ones)
            plsc.addupdate_scatter(hist, [lax.shift_right_logical(w, 16)], ones)

    pltpu.sync_copy(hist.at[pl.ds(0, NCOMB)],
                    outc_hbm.at[pl.ds(wid * NCOMB, NCOMB)])
    pltpu.sync_copy(hist.at[pl.ds(NCOMB, NEYE)],
                    oute_hbm.at[pl.ds(wid * NEYE, NEYE)])


def _tables_body(comb_ref, eye_ref, tb_ref, te_ref):
    hc = jnp.sum(comb_ref[...], axis=0)
    he = jnp.sum(eye_ref[...], axis=0)[:, 256:512]
    combos = [hc[:, b * 256:(b + 1) * 256] for b in range(8)]
    hists = [
        combos[1] + combos[3] + combos[5] + combos[7],
        combos[2] + combos[3] + combos[6] + combos[7],
        combos[4] + combos[5] + combos[6] + combos[7],
        he,
    ]
    jj = lax.broadcasted_iota(I32, (256, 256), 0)
    ii = lax.broadcasted_iota(I32, (256, 256), 1)
    tri = (jj <= ii).astype(F32)
    tabs = []
    for h in hists:
        cs = lax.dot_general(h, tri, (((1,), (0,)), ((), ())),
                             precision=lax.Precision.HIGHEST,
                             preferred_element_type=F32)
        csn = cs / jnp.maximum(cs[:, 255:256], 1.0)
        tm = []
        for img in range(4):
            a = csn[img * 6: img * 6 + 3]
            b = csn[img * 6 + 3: img * 6 + 6]
            aa = jnp.broadcast_to(a[:, :, None], (3, 256, 256))
            bb = jnp.broadcast_to(b[:, None, :], (3, 256, 256))
            cnt = jnp.sum((bb < aa).astype(F32), axis=2)
            tm.append(jnp.clip(cnt, 0.0, 255.0).astype(I32))
        tabs.append(tm)
    for img in range(4):
        tb_ref[img] = tabs[0][img] | (tabs[1][img] << 8) | (tabs[2][img] << 16)
        te_ref[img] = tabs[3][img]


def _tables(comb, eye):
    return pl.pallas_call(
        _tables_body,
        out_shape=[
            jax.ShapeDtypeStruct((4, 3, 256), I32),
            jax.ShapeDtypeStruct((4, 3, 256), I32),
        ],
    )(comb, eye)


@functools.cache
def _gather_sc_kernel():
    mesh = plsc.VectorSubcoreMesh(core_axis_name="c", subcore_axis_name="s")
    return pl.kernel(
        _gather_sc_body,
        out_type=jax.ShapeDtypeStruct((4 * 3 * NPIX,), I32),
        mesh=mesh,
        scratch_types=[
            pltpu.VMEM((2 * 4 * 3 * 256,), I32),
            pltpu.VMEM((CHUNK,), I32),
            pltpu.VMEM((CHUNK,), I32),
            pltpu.VMEM((CHUNK,), I32),
            pltpu.VMEM((CHUNK,), I32),
            pltpu.VMEM((CHUNK,), I32),
        ],
        compiler_params=pltpu.CompilerParams(needs_layout_passes=False),
    )


def _gather_sc_body(tab_hbm, p1_hbm, p2_hbm, out_hbm, tab, buf1, buf2, ob0, ob1, ob2):
    wid = lax.axis_index("s") * NC + lax.axis_index("c")
    base = wid * CHUNK
    obufs = [ob0, ob1, ob2]
    pltpu.sync_copy(tab_hbm, tab)
    for img in range(4):
        pltpu.sync_copy(p1_hbm.at[pl.ds(img * NPIX + base, CHUNK)], buf1)
        pltpu.sync_copy(p2_hbm.at[pl.ds(img * NPIX + base, CHUNK)], buf2)

        @plsc.parallel_loop(0, CHUNK // 16, unroll=4)
        def _(i):
            w1 = buf1[pl.ds(i * 16, 16)]
            w2 = buf2[pl.ds(i * 16, 16)]
            for ch in range(3):
                vc = (w1 >> (8 * ch)) & 255
                vec = (w2 >> (8 * ch)) & 255
                g = plsc.load_gather(tab, [vc + (img * 3 + ch) * 256])
                ge = plsc.load_gather(tab, [vec + (3072 + (img * 3 + ch) * 256)])
                obufs[ch][pl.ds(i * 16, 16)] = g | (ge << 24)

        for ch in range(3):
            pltpu.sync_copy(obufs[ch], out_hbm.at[pl.ds((img * 3 + ch) * NPIX + base, CHUNK)])


def _compose_body(src_ref, tgt_ref, ms_ref, be_ref, mt_ref, lms_s_ref, lms_r_ref, out_ref):
    m_skin = ms_ref[0, 1]
    m_neck = ms_ref[0, 4]
    m_lip = ms_ref[0, 0]
    m_eyec = jnp.clip(ms_ref[0, 2] + ms_ref[0, 3], 0.0, 1.0)
    bew = be_ref[0]

    def mean_delta(lo, hi):
        n = float(hi - lo)

        def acc(j, c):
            return (c[0] + lms_s_ref[0, j, 0], c[1] + lms_s_ref[0, j, 1],
                    c[2] + lms_r_ref[0, j, 0], c[3] + lms_r_ref[0, j, 1])
        s0, s1, r0, r1 = lax.fori_loop(lo, hi, acc, (0.0, 0.0, 0.0, 0.0))
        d0 = (s0 / n - r0 / n + 1024.5).astype(I32) - 1024
        d1 = (s1 / n - r1 / n + 1024.5).astype(I32) - 1024
        return jnp.mod(d0, H), jnp.mod(d1, W)

    deltas = [mean_delta(48, 68), mean_delta(0, 68), mean_delta(36, 48)]
    regions = [(0.1, m_lip), (0.3, m_skin), (0.8, m_eyec)]

    for c in range(3):
        acc = src_ref[0, c]
        wc = mt_ref[0, c]
        for m, mk in ((0, m_skin), (1, m_neck), (2, m_lip)):
            t = ((wc >> (8 * m)) & 255).astype(F32) / 255.0 * 2.0 - 1.0
            acc = (1.0 - mk) * acc + mk * t
        t = ((wc >> 24) & 255).astype(F32) / 255.0 * 2.0 - 1.0
        acc = (1.0 - bew) * acc + bew * t
        tgt_c = tgt_ref[0, c]
        for (alpha, mk), (d0, d1) in zip(regions, deltas):
            rolled = pltpu.roll(pltpu.roll(tgt_c, d0, 0), d1, 1)
            wgt = alpha * mk
            acc = (1.0 - wgt) * acc + wgt * rolled
        out_ref[0, c] = acc


def _compose(sources, targets, mask_srcs, be, matched, lms_srcs, lms_tars):
    return pl.pallas_call(
        _compose_body,
        grid=(4,),
        in_specs=[
            pl.BlockSpec((1, 3, H, W), lambda i: (i, 0, 0, 0)),
            pl.BlockSpec((1, 3, H, W), lambda i: (i, 0, 0, 0)),
            pl.BlockSpec((1, 5, H, W), lambda i: (i, 0, 0, 0)),
            pl.BlockSpec((1, H, W), lambda i: (i, 0, 0)),
            pl.BlockSpec((1, 3, H, W), lambda i: (i, 0, 0, 0)),
            pl.BlockSpec((1, 68, 2), lambda i: (i, 0, 0), memory_space=pltpu.SMEM),
            pl.BlockSpec((1, 68, 2), lambda i: (i, 0, 0), memory_space=pltpu.SMEM),
        ],
        out_specs=pl.BlockSpec((1, 3, H, W), lambda i: (i, 0, 0, 0)),
        out_shape=jax.ShapeDtypeStruct((4, 3, H, W), F32),
    )(sources, targets, mask_srcs, be, matched, lms_srcs, lms_tars)


def kernel(sources, targets, mask_srcs, mask_tars, lms_srcs, lms_tars):
    hk, p1, p2, be = _prep(sources, targets, mask_srcs, mask_tars)
    p1f = p1.reshape(4 * NPIX)
    p2f = p2.reshape(4 * NPIX)
    hcomb, heye = _hist_sc_kernel()(hk.reshape(-1))
    comb = hcomb.reshape(NW, 24, 2048)
    eye = heye.reshape(NW, 24, 512)
    tb, te = _tables(comb, eye)
    tabflat = jnp.concatenate([tb.reshape(-1), te.reshape(-1)])
    matched = _gather_sc_kernel()(tabflat, p1f, p2f)
    return _compose(sources, targets, mask_srcs, be,
                    matched.reshape(4, 3, H, W), lms_srcs, lms_tars)
